# baseline (speedup 1.0000x reference)
import functools

import jax
import jax.numpy as jnp
from jax import lax
from jax.experimental import pallas as pl
from jax.experimental.pallas import tpu as pltpu
from jax.experimental.pallas import tpu_sc as plsc

N_IN = 2
EMB = 128
B = 16384
D = 128
ROWS = 4096

NC = 2
NS = 16
NW = NC * NS
ROWS_W = B // NW
CH = 128
N_CHUNK = ROWS_W // CH


def _tc_body(x1, x2, w1, b1, w2, b2, w3, b3, sel, wb,
             out_ref, out1_ref, out2_ref):
    a10 = x1[0]
    a11 = x1[1]
    a20 = x2[0]
    a21 = x2[1]

    def dock(x, w, b, i):
        return jax.nn.relu(
            jnp.dot(x.astype(jnp.bfloat16), w[i].astype(jnp.bfloat16),
                    preferred_element_type=jnp.float32) + b[i:i + 1, :])

    s1 = sel[0:1, :]
    o1 = jnp.where(s1 == 0, dock(a10, w1, b1, 0), dock(a11, w1, b1, 1))
    s2 = sel[1:2, :]
    o2 = jnp.where(s2 == 0, dock(a10, w2, b2, 0), dock(a11, w2, b2, 1))

    ws = a20 * wb[0:1, :] + a21 * wb[1:2, :]

    s3 = sel[2:3, :]
    m = jnp.where(s3 == 0, dock(a20, w3, b3, 0),
        jnp.where(s3 == 1, dock(a21, w3, b3, 1),
        jnp.where(s3 == 2, dock(o1, w3, b3, 2),
        jnp.where(s3 == 3, dock(o2, w3, b3, 3), dock(ws, w3, b3, 4)))))

    out_ref[...] = m
    out1_ref[...] = o1
    out2_ref[...] = o2


def _sc_ws_kernel(x2_hbm, wlanes_hbm, out_hbm, a_v, b_v, o_v, w_v, sem):
    wid = lax.axis_index("s") * NC + lax.axis_index("c")
    base = wid * ROWS_W
    pltpu.sync_copy(wlanes_hbm, w_v)
    w0 = w_v[0]
    w1 = w_v[1]

    for t in range(N_CHUNK):
        r0 = base + t * CH
        pltpu.async_copy(x2_hbm.at[0, pl.ds(r0, CH)], a_v, sem)
        cp = pltpu.async_copy(x2_hbm.at[1, pl.ds(r0, CH)], b_v, sem)
        cp.wait()
        cp.wait()

        def row(r, carry):
            for c in range(D // 16):
                sl = pl.ds(c * 16, 16)
                o_v[r, sl] = a_v[r, sl] * w0 + b_v[r, sl] * w1
            return carry

        lax.fori_loop(0, CH, row, 0, unroll=2)
        pltpu.sync_copy(o_v, out_hbm.at[pl.ds(r0, CH)])


def kernel(outputs1, outputs2, available, W1, b1, W2, b2, W3, b3, ws_w):
    del available

    k = jax.random.key(42)
    k1, k2, k3 = jax.random.split(k, 3)
    ones12 = jnp.ones((1, N_IN), dtype=jnp.float32)
    p12 = ones12 / jnp.sum(ones12, axis=-1, keepdims=True)
    idx1 = jax.random.categorical(k1, jnp.log(p12), shape=(1, EMB))
    idx2 = jax.random.categorical(k2, jnp.log(p12), shape=(1, EMB))
    avail = jnp.ones((1, N_IN + 3), dtype=jnp.float32)
    p3 = avail / jnp.sum(avail, axis=-1, keepdims=True)
    idx3 = jax.random.categorical(k3, jnp.log(p3), shape=(1, EMB))
    sel = jnp.concatenate([idx1, idx2, idx3], axis=0).astype(jnp.int32)

    w = ws_w * avail[0, :N_IN]
    w = w / jnp.sum(w)
    wb = jnp.broadcast_to(w[:, None], (N_IN, EMB)).astype(jnp.float32)
    wlanes = jnp.broadcast_to(w[:, None], (N_IN, 16)).astype(jnp.float32)

    grid = (B // ROWS,)
    row_spec = pl.BlockSpec((ROWS, D), lambda i: (i, 0))
    xin_spec = pl.BlockSpec((N_IN, ROWS, D), lambda i: (0, i, 0))
    full = lambda shape: pl.BlockSpec(shape, lambda i: (0,) * len(shape))

    out, out1, out2 = pl.pallas_call(
        _tc_body,
        grid=grid,
        in_specs=[
            xin_spec, xin_spec,
            full((N_IN, D, EMB)), full((N_IN, EMB)),
            full((N_IN, D, EMB)), full((N_IN, EMB)),
            full((N_IN + 3, D, EMB)), full((N_IN + 3, EMB)),
            full((3, EMB)), full((N_IN, EMB)),
        ],
        out_specs=(row_spec, row_spec, row_spec),
        out_shape=tuple(
            jax.ShapeDtypeStruct((B, EMB), jnp.float32) for _ in range(3)),
    )(outputs1, outputs2, W1, b1, W2, b2, W3, b3, sel, wb)

    mesh = plsc.VectorSubcoreMesh(core_axis_name="c", subcore_axis_name="s")
    wsout = pl.kernel(
        _sc_ws_kernel,
        mesh=mesh,
        out_type=jax.ShapeDtypeStruct((B, D), jnp.float32),
        scratch_types=[
            pltpu.VMEM((CH, D), jnp.float32),
            pltpu.VMEM((CH, D), jnp.float32),
            pltpu.VMEM((CH, D), jnp.float32),
            pltpu.VMEM((N_IN, 16), jnp.float32),
            pltpu.SemaphoreType.DMA,
        ],
    )(outputs2, wlanes)

    return (out, (out1, out2, wsout))


# speedup vs baseline: 1.1328x; 1.1328x over previous
import functools

import jax
import jax.numpy as jnp
from jax import lax
from jax.experimental import pallas as pl
from jax.experimental.pallas import tpu as pltpu
from jax.experimental.pallas import tpu_sc as plsc

N_IN = 2
EMB = 128
B = 16384
D = 128
ROWS = 4096

NC = 2
NS = 16
NW = NC * NS
ROWS_W = B // NW
CH = 128
N_CHUNK = ROWS_W // CH


def _tc_body(x1, x2, w1, b1, w2, b2, w3, b3, sel, wb,
             out_ref, out1_ref, out2_ref):
    a10 = x1[0]
    a11 = x1[1]
    a20 = x2[0]
    a21 = x2[1]

    def dock(x, w, b, i):
        return jax.nn.relu(
            jnp.dot(x.astype(jnp.bfloat16), w[i].astype(jnp.bfloat16),
                    preferred_element_type=jnp.float32) + b[i:i + 1, :])

    s1 = sel[0:1, :]
    o1 = jnp.where(s1 == 0, dock(a10, w1, b1, 0), dock(a11, w1, b1, 1))
    s2 = sel[1:2, :]
    o2 = jnp.where(s2 == 0, dock(a10, w2, b2, 0), dock(a11, w2, b2, 1))

    ws = a20 * wb[0:1, :] + a21 * wb[1:2, :]

    s3 = sel[2:3, :]
    m = jnp.where(s3 == 0, dock(a20, w3, b3, 0),
        jnp.where(s3 == 1, dock(a21, w3, b3, 1),
        jnp.where(s3 == 2, dock(o1, w3, b3, 2),
        jnp.where(s3 == 3, dock(o2, w3, b3, 3), dock(ws, w3, b3, 4)))))

    out_ref[...] = m
    out1_ref[...] = o1
    out2_ref[...] = o2


def _sc_ws_kernel(x2_hbm, wlanes_hbm, out_hbm,
                  a0, a1, b0, b1, o0, o1, w_v,
                  sin0, sin1, sout0, sout1):
    wid = lax.axis_index("s") * NC + lax.axis_index("c")
    base = wid * ROWS_W
    pltpu.sync_copy(wlanes_hbm, w_v)
    w0 = w_v[0]
    w1 = w_v[1]

    abuf = (a0, a1)
    bbuf = (b0, b1)
    obuf = (o0, o1)
    sin = (sin0, sin1)
    sout = (sout0, sout1)

    pltpu.async_copy(x2_hbm.at[0, pl.ds(base, CH)], a0, sin0)
    last_in = pltpu.async_copy(x2_hbm.at[1, pl.ds(base, CH)], b0, sin0)
    out_cp = [None, None]

    for t in range(N_CHUNK):
        cur = t % 2
        nxt = 1 - cur
        if t + 1 < N_CHUNK:
            r1 = base + (t + 1) * CH
            pltpu.async_copy(x2_hbm.at[0, pl.ds(r1, CH)], abuf[nxt], sin[nxt])
            last_in = pltpu.async_copy(
                x2_hbm.at[1, pl.ds(r1, CH)], bbuf[nxt], sin[nxt])
        cp = pltpu.make_async_copy(x2_hbm.at[0, pl.ds(base, CH)],
                                   abuf[cur], sin[cur])
        cp.wait()
        cp.wait()
        if out_cp[cur] is not None:
            out_cp[cur].wait()

        ov = obuf[cur]
        av = abuf[cur]
        bv = bbuf[cur]

        def row(r, carry):
            for c in range(D // 16):
                sl = pl.ds(c * 16, 16)
                ov[r, sl] = av[r, sl] * w0 + bv[r, sl] * w1
            return carry

        lax.fori_loop(0, CH, row, 0, unroll=4)
        out_cp[cur] = pltpu.async_copy(
            ov, out_hbm.at[pl.ds(base + t * CH, CH)], sout[cur])

    out_cp[0].wait()
    out_cp[1].wait()
    del last_in


def kernel(outputs1, outputs2, available, W1, b1, W2, b2, W3, b3, ws_w):
    del available

    k = jax.random.key(42)
    k1, k2, k3 = jax.random.split(k, 3)
    ones12 = jnp.ones((1, N_IN), dtype=jnp.float32)
    p12 = ones12 / jnp.sum(ones12, axis=-1, keepdims=True)
    idx1 = jax.random.categorical(k1, jnp.log(p12), shape=(1, EMB))
    idx2 = jax.random.categorical(k2, jnp.log(p12), shape=(1, EMB))
    avail = jnp.ones((1, N_IN + 3), dtype=jnp.float32)
    p3 = avail / jnp.sum(avail, axis=-1, keepdims=True)
    idx3 = jax.random.categorical(k3, jnp.log(p3), shape=(1, EMB))
    sel = jnp.concatenate([idx1, idx2, idx3], axis=0).astype(jnp.int32)

    w = ws_w * avail[0, :N_IN]
    w = w / jnp.sum(w)
    wb = jnp.broadcast_to(w[:, None], (N_IN, EMB)).astype(jnp.float32)
    wlanes = jnp.broadcast_to(w[:, None], (N_IN, 16)).astype(jnp.float32)

    grid = (B // ROWS,)
    row_spec = pl.BlockSpec((ROWS, D), lambda i: (i, 0))
    xin_spec = pl.BlockSpec((N_IN, ROWS, D), lambda i: (0, i, 0))
    full = lambda shape: pl.BlockSpec(shape, lambda i: (0,) * len(shape))

    out, out1, out2 = pl.pallas_call(
        _tc_body,
        grid=grid,
        in_specs=[
            xin_spec, xin_spec,
            full((N_IN, D, EMB)), full((N_IN, EMB)),
            full((N_IN, D, EMB)), full((N_IN, EMB)),
            full((N_IN + 3, D, EMB)), full((N_IN + 3, EMB)),
            full((3, EMB)), full((N_IN, EMB)),
        ],
        out_specs=(row_spec, row_spec, row_spec),
        out_shape=tuple(
            jax.ShapeDtypeStruct((B, EMB), jnp.float32) for _ in range(3)),
    )(outputs1, outputs2, W1, b1, W2, b2, W3, b3, sel, wb)

    mesh = plsc.VectorSubcoreMesh(core_axis_name="c", subcore_axis_name="s")
    wsout = pl.kernel(
        _sc_ws_kernel,
        mesh=mesh,
        out_type=jax.ShapeDtypeStruct((B, D), jnp.float32),
        scratch_types=(
            [pltpu.VMEM((CH, D), jnp.float32) for _ in range(6)]
            + [pltpu.VMEM((N_IN, 16), jnp.float32)]
            + [pltpu.SemaphoreType.DMA for _ in range(4)]
        ),
    )(outputs2, wlanes)

    return (out, (out1, out2, wsout))


# speedup vs baseline: 1.5773x; 1.3924x over previous
import functools

import jax
import jax.numpy as jnp
from jax.experimental import pallas as pl

N_IN = 2
EMB = 128
B = 16384
D = 128
ROWS = 4096


def _fused_body(x1, x2, w1, b1, w2, b2, w3, b3, sel, wb,
                out_ref, out1_ref, out2_ref, ws_ref):
    a10 = x1[0]
    a11 = x1[1]
    a20 = x2[0]
    a21 = x2[1]

    def dock(x, w, b, i):
        return jax.nn.relu(
            jnp.dot(x.astype(jnp.bfloat16), w[i].astype(jnp.bfloat16),
                    preferred_element_type=jnp.float32) + b[i:i + 1, :])

    s1 = sel[0:1, :]
    o1 = jnp.where(s1 == 0, dock(a10, w1, b1, 0), dock(a11, w1, b1, 1))
    s2 = sel[1:2, :]
    o2 = jnp.where(s2 == 0, dock(a10, w2, b2, 0), dock(a11, w2, b2, 1))

    ws = a20 * wb[0:1, :] + a21 * wb[1:2, :]

    s3 = sel[2:3, :]
    m = jnp.where(s3 == 0, dock(a20, w3, b3, 0),
        jnp.where(s3 == 1, dock(a21, w3, b3, 1),
        jnp.where(s3 == 2, dock(o1, w3, b3, 2),
        jnp.where(s3 == 3, dock(o2, w3, b3, 3), dock(ws, w3, b3, 4)))))

    out_ref[...] = m
    out1_ref[...] = o1
    out2_ref[...] = o2
    ws_ref[...] = ws


def kernel(outputs1, outputs2, available, W1, b1, W2, b2, W3, b3, ws_w):
    del available

    k = jax.random.key(42)
    k1, k2, k3 = jax.random.split(k, 3)
    ones12 = jnp.ones((1, N_IN), dtype=jnp.float32)
    p12 = ones12 / jnp.sum(ones12, axis=-1, keepdims=True)
    idx1 = jax.random.categorical(k1, jnp.log(p12), shape=(1, EMB))
    idx2 = jax.random.categorical(k2, jnp.log(p12), shape=(1, EMB))
    avail = jnp.ones((1, N_IN + 3), dtype=jnp.float32)
    p3 = avail / jnp.sum(avail, axis=-1, keepdims=True)
    idx3 = jax.random.categorical(k3, jnp.log(p3), shape=(1, EMB))
    sel = jnp.concatenate([idx1, idx2, idx3], axis=0).astype(jnp.int32)

    w = ws_w * avail[0, :N_IN]
    w = w / jnp.sum(w)
    wb = jnp.broadcast_to(w[:, None], (N_IN, EMB)).astype(jnp.float32)

    grid = (B // ROWS,)
    row_spec = pl.BlockSpec((ROWS, D), lambda i: (i, 0))
    xin_spec = pl.BlockSpec((N_IN, ROWS, D), lambda i: (0, i, 0))
    full = lambda shape: pl.BlockSpec(shape, lambda i: (0,) * len(shape))

    out_shapes = tuple(
        jax.ShapeDtypeStruct((B, EMB), jnp.float32) for _ in range(4))

    out, out1, out2, wsout = pl.pallas_call(
        _fused_body,
        grid=grid,
        in_specs=[
            xin_spec, xin_spec,
            full((N_IN, D, EMB)), full((N_IN, EMB)),
            full((N_IN, D, EMB)), full((N_IN, EMB)),
            full((N_IN + 3, D, EMB)), full((N_IN + 3, EMB)),
            full((3, EMB)), full((N_IN, EMB)),
        ],
        out_specs=(row_spec, row_spec, row_spec, row_spec),
        out_shape=out_shapes,
    )(outputs1, outputs2, W1, b1, W2, b2, W3, b3, sel, wb)

    return (out, (out1, out2, wsout))
